# Initial kernel scaffold; baseline (speedup 1.0000x reference)
import jax
import jax.numpy as jnp
from jax.experimental import pallas as pl


def kernel(data, ivectors):
    raise NotImplementedError("write your pallas kernel here")


# speedup vs baseline: 1.6824x; 1.6824x over previous
import functools

import jax
import jax.numpy as jnp
from jax import lax
from jax.experimental import pallas as pl
from jax.experimental.pallas import tpu as pltpu
from jax.experimental.pallas import tpu_sc as plsc

_ITEM_NUM = 1000000
_EMBED_DIM = 64
_BATCH = 16384
_HIST = 50

_NC = 2
_NS = 16
_NW = _NC * _NS
_B = _BATCH * _HIST
_BPW = _B // _NW
_C = 128
_NCHUNK = _BPW // _C


def _sc_gather(table, idx):
    mesh = plsc.VectorSubcoreMesh(core_axis_name="c", subcore_axis_name="s")

    @functools.partial(
        pl.kernel,
        out_type=jax.ShapeDtypeStruct((_B, _EMBED_DIM), jnp.float32),
        mesh=mesh,
        scratch_types=[
            pltpu.VMEM((_BPW,), jnp.int32),
            pltpu.VMEM((_C, _EMBED_DIM), jnp.float32),
            pltpu.SemaphoreType.DMA,
        ],
        compiler_params=pltpu.CompilerParams(use_tc_tiling_on_sc=False),
    )
    def k(table_hbm, idx_hbm, out_hbm, idx_v, rows_v, sem):
        wid = lax.axis_index("s") * _NC + lax.axis_index("c")
        base = wid * _BPW
        pltpu.sync_copy(idx_hbm.at[pl.ds(base, _BPW)], idx_v)

        def body(c, carry):
            off = c * _C
            pltpu.async_copy(
                table_hbm.at[idx_v.at[pl.ds(off, _C)]], rows_v, sem
            ).wait()
            pltpu.sync_copy(rows_v, out_hbm.at[pl.ds(base + off, _C)])
            return carry

        lax.fori_loop(0, _NCHUNK, body, 0)

    return k(table, idx)


def kernel(data, ivectors):
    flat = data.reshape(-1).astype(jnp.int32)
    out = _sc_gather(ivectors, flat)
    return out.reshape(_BATCH, _HIST, _EMBED_DIM)


# speedup vs baseline: 1.8796x; 1.1172x over previous
import functools

import jax
import jax.numpy as jnp
from jax import lax
from jax.experimental import pallas as pl
from jax.experimental.pallas import tpu as pltpu
from jax.experimental.pallas import tpu_sc as plsc

_ITEM_NUM = 1000000
_EMBED_DIM = 64
_BATCH = 16384
_HIST = 50

_NC = 2
_NS = 16
_NW = _NC * _NS
_B = _BATCH * _HIST
_BPW = _B // _NW
_C = 128
_NCHUNK = _BPW // _C
_NBUF = 4


def _sc_gather(table, idx):
    mesh = plsc.VectorSubcoreMesh(core_axis_name="c", subcore_axis_name="s")

    @functools.partial(
        pl.kernel,
        out_type=jax.ShapeDtypeStruct((_B, _EMBED_DIM), jnp.float32),
        mesh=mesh,
        scratch_types=[
            pltpu.VMEM((_BPW,), jnp.int32),
            pltpu.VMEM((_NBUF, _C, _EMBED_DIM), jnp.float32),
            pltpu.SemaphoreType.DMA,
            pltpu.SemaphoreType.DMA,
        ],
        compiler_params=pltpu.CompilerParams(use_tc_tiling_on_sc=False),
    )
    def k(table_hbm, idx_hbm, out_hbm, idx_v, rows, gsem, ssem):
        wid = lax.axis_index("s") * _NC + lax.axis_index("c")
        base = wid * _BPW
        pltpu.sync_copy(idx_hbm.at[pl.ds(base, _BPW)], idx_v)

        def gather(c, b):
            pltpu.async_copy(
                table_hbm.at[idx_v.at[pl.ds(c * _C, _C)]], rows.at[b], gsem
            )

        for b in range(_NBUF - 1):
            gather(b, b)

        def body(o, carry):
            for u in range(_NBUF):
                c = o * _NBUF + u
                b = u
                pltpu.make_async_copy(
                    table_hbm.at[pl.ds(0, _C)], rows.at[b], gsem
                ).wait()
                pltpu.async_copy(
                    rows.at[b], out_hbm.at[pl.ds(base + c * _C, _C)], ssem
                )
                @pl.when(c >= 1)
                def _():
                    pltpu.make_async_copy(
                        table_hbm.at[pl.ds(0, _C)],
                        rows.at[(b - 1) % _NBUF],
                        ssem,
                    ).wait()

                @pl.when(c + _NBUF - 1 < _NCHUNK)
                def _():
                    gather(c + _NBUF - 1, (b - 1) % _NBUF)

            return carry

        lax.fori_loop(0, _NCHUNK // _NBUF, body, 0)
        pltpu.make_async_copy(
            table_hbm.at[pl.ds(0, _C)], rows.at[_NBUF - 1], ssem
        ).wait()

    return k(table, idx)


def kernel(data, ivectors):
    flat = data.reshape(-1).astype(jnp.int32)
    out = _sc_gather(ivectors, flat)
    return out.reshape(_BATCH, _HIST, _EMBED_DIM)
